# baseline (speedup 1.0000x reference)
import jax
import jax.numpy as jnp
from jax import lax
from jax.experimental import pallas as pl
from jax.experimental.pallas import tpu as pltpu

_N = 8192
_D = 256
_CH = 1024
_NCH = _N // _CH


def _body(x_ref, w_hbm, out_ref, buf0, buf1, sem0, sem1):
    bufs = (buf0, buf1)
    sems = (sem0, sem1)

    def chunk_copy(ch, k):
        return pltpu.make_async_copy(
            w_hbm.at[pl.ds(ch * _CH, _CH)], bufs[k], sems[k])

    chunk_copy(0, 0).start()
    chunk_copy(1, 1).start()

    xb = x_ref[...]
    rowid0 = lax.broadcasted_iota(jnp.int32, (_CH, 1), 0)
    runmin_v = jnp.full((_CH, 1), jnp.inf, jnp.float32)
    runidx_v = jnp.zeros((_CH, 1), jnp.int32)

    for ch in range(_NCH):
        k = ch % 2
        chunk_copy(ch, k).wait()
        w = bufs[k][...]
        d = w - xb
        dist2 = jnp.sum(d * d, axis=1, keepdims=True)
        if ch + 2 < _NCH:
            chunk_copy(ch + 2, k).start()
        better = dist2 < runmin_v
        runmin_v = jnp.where(better, dist2, runmin_v)
        runidx_v = jnp.where(better, rowid0 + ch * _CH, runidx_v)

    m = jnp.min(runmin_v)
    cand = jnp.where(runmin_v == m, runidx_v, jnp.int32(_N))
    win = jnp.min(cand)
    rows = lax.broadcasted_iota(jnp.int32, (_N // 128, 128), 0)
    cols = lax.broadcasted_iota(jnp.int32, (_N // 128, 128), 1)
    out_ref[...] = ((rows * 128 + cols) == win).astype(jnp.float32)


_distance_argmin = pl.pallas_call(
    _body,
    in_specs=[
        pl.BlockSpec((1, _D), lambda: (0, 0)),
        pl.BlockSpec(memory_space=pltpu.MemorySpace.HBM),
    ],
    out_specs=pl.BlockSpec((_N // 128, 128), lambda: (0, 0)),
    out_shape=jax.ShapeDtypeStruct((_N // 128, 128), jnp.float32),
    scratch_shapes=[
        pltpu.VMEM((_CH, _D), jnp.float32),
        pltpu.VMEM((_CH, _D), jnp.float32),
        pltpu.SemaphoreType.DMA,
        pltpu.SemaphoreType.DMA,
    ],
)


def kernel(x, reward, weights, thresholds):
    act = _distance_argmin(x.reshape(1, _D), weights)
    return act.reshape(_N)


# speedup vs baseline: 1.4113x; 1.4113x over previous
import jax
import jax.numpy as jnp
from jax import lax
from jax.experimental import pallas as pl
from jax.experimental.pallas import tpu as pltpu

_N = 8192
_D = 256
_CH = 512
_NCH = _N // _CH


def _body(x_ref, w_hbm, out_ref, bufs, sems):
    def chunk_copy(ch):
        return pltpu.make_async_copy(
            w_hbm.at[pl.ds(ch * _CH, _CH)], bufs.at[ch], sems.at[ch])

    for ch in range(_NCH):
        chunk_copy(ch).start()

    xb = x_ref[...]
    rowid0 = lax.broadcasted_iota(jnp.int32, (_CH, 1), 0)
    runmin_v = jnp.full((_CH, 1), jnp.inf, jnp.float32)
    runidx_v = jnp.zeros((_CH, 1), jnp.int32)

    for ch in range(_NCH):
        chunk_copy(ch).wait()
        w = bufs[ch]
        d = w - xb
        dist2 = jnp.sum(d * d, axis=1, keepdims=True)
        better = dist2 < runmin_v
        runmin_v = jnp.where(better, dist2, runmin_v)
        runidx_v = jnp.where(better, rowid0 + ch * _CH, runidx_v)

    m = jnp.min(runmin_v)
    cand = jnp.where(runmin_v == m, runidx_v, jnp.int32(_N))
    win = jnp.min(cand)
    rows = lax.broadcasted_iota(jnp.int32, (_N // 128, 128), 0)
    cols = lax.broadcasted_iota(jnp.int32, (_N // 128, 128), 1)
    out_ref[...] = ((rows * 128 + cols) == win).astype(jnp.float32)


_distance_argmin = pl.pallas_call(
    _body,
    in_specs=[
        pl.BlockSpec((1, _D), lambda: (0, 0)),
        pl.BlockSpec(memory_space=pltpu.MemorySpace.HBM),
    ],
    out_specs=pl.BlockSpec((_N // 128, 128), lambda: (0, 0)),
    out_shape=jax.ShapeDtypeStruct((_N // 128, 128), jnp.float32),
    scratch_shapes=[
        pltpu.VMEM((_NCH, _CH, _D), jnp.float32),
        pltpu.SemaphoreType.DMA((_NCH,)),
    ],
)


def kernel(x, reward, weights, thresholds):
    act = _distance_argmin(x.reshape(1, _D), weights)
    return act.reshape(_N)


# speedup vs baseline: 1.4486x; 1.0264x over previous
import jax
import jax.numpy as jnp
from jax import lax
from jax.experimental import pallas as pl
from jax.experimental.pallas import tpu as pltpu

_N = 8192
_D = 256
_BLK = 4096
_G = _N // _BLK


def _body(x_ref, w_ref, out_ref, runmin_s, runidx_s):
    i = pl.program_id(0)

    @pl.when(i == 0)
    def _init():
        runmin_s[0] = jnp.float32(jnp.inf)
        runidx_s[0] = jnp.int32(0)

    w = w_ref[...]
    xb = x_ref[...]
    d = w - xb
    dist2 = jnp.sum(d * d, axis=1, keepdims=True)
    m = jnp.min(dist2)
    rowid = lax.broadcasted_iota(jnp.int32, (_BLK, 1), 0) + i * _BLK
    cand = jnp.where(dist2 == m, rowid, jnp.int32(_N))
    li = jnp.min(cand)

    better = m < runmin_s[0]
    runmin_s[0] = jnp.where(better, m, runmin_s[0])
    runidx_s[0] = jnp.where(better, li, runidx_s[0])

    @pl.when(i == _G - 1)
    def _emit():
        win = runidx_s[0]
        rows = lax.broadcasted_iota(jnp.int32, (_N // 128, 128), 0)
        cols = lax.broadcasted_iota(jnp.int32, (_N // 128, 128), 1)
        out_ref[...] = ((rows * 128 + cols) == win).astype(jnp.float32)


_distance_argmin = pl.pallas_call(
    _body,
    grid=(_G,),
    in_specs=[
        pl.BlockSpec((1, _D), lambda i: (0, 0)),
        pl.BlockSpec((_BLK, _D), lambda i: (i, 0)),
    ],
    out_specs=pl.BlockSpec((_N // 128, 128), lambda i: (0, 0)),
    out_shape=jax.ShapeDtypeStruct((_N // 128, 128), jnp.float32),
    scratch_shapes=[
        pltpu.SMEM((1,), jnp.float32),
        pltpu.SMEM((1,), jnp.int32),
    ],
)


def kernel(x, reward, weights, thresholds):
    act = _distance_argmin(x.reshape(1, _D), weights)
    return act.reshape(_N)
